# Initial kernel scaffold; baseline (speedup 1.0000x reference)
import jax
import jax.numpy as jnp
from jax.experimental import pallas as pl


def kernel(src_seq, src_pos, emb, W_ih, W_hh, b_ih, b_hh, W1, b1, W2, b2):
    raise NotImplementedError("write your pallas kernel here")


# speedup vs baseline: 2.6724x; 2.6724x over previous
import jax
import jax.numpy as jnp
from jax import lax
from jax.experimental import pallas as pl
from jax.experimental.pallas import tpu as pltpu
from jax.experimental.pallas import tpu_sc as plsc

B, T, V, D, H, F = 1024, 50, 1000, 512, 512, 2048
G = 4 * H
OUT_PAD = 128

_NC, _NS = 2, 16
_NW = _NC * _NS
_ROWS_PER_W = (T * B) // _NW
_CHUNK = 32
_NCHUNK = _ROWS_PER_W // _CHUNK


def _proj_body(emb_ref, w_ref, b_ref, out_ref):
    out_ref[...] = (
        jnp.dot(emb_ref[...], w_ref[...], preferred_element_type=jnp.float32)
        + b_ref[...]
    )


def _gather_body(table_hbm, idx_hbm, out_hbm, idx_v, rows_v, sem):
    wid = lax.axis_index("s") * _NC + lax.axis_index("c")
    base = wid * _ROWS_PER_W
    pltpu.sync_copy(idx_hbm.at[pl.ds(base, _ROWS_PER_W)], idx_v)

    def chunk(ch, carry):
        r0 = ch * _CHUNK
        pltpu.async_copy(
            table_hbm.at[idx_v.at[pl.ds(r0, _CHUNK)]], rows_v, sem
        ).wait()
        pltpu.sync_copy(rows_v, out_hbm.at[pl.ds(base + r0, _CHUNK)])
        return carry

    lax.fori_loop(0, _NCHUNK, chunk, 0)


def _lstm_body(x_ref, whh_ref, w1_ref, b1_ref, w2_ref, b2_ref,
               out_ref, h_ref, c_ref):
    t = pl.program_id(0)

    @pl.when(t == 0)
    def _():
        h_ref[...] = jnp.zeros_like(h_ref)
        c_ref[...] = jnp.zeros_like(c_ref)

    gates = x_ref[0] + jnp.dot(
        h_ref[...], whh_ref[...], preferred_element_type=jnp.float32
    )
    i = jax.nn.sigmoid(gates[:, 0:H])
    f = jax.nn.sigmoid(gates[:, H:2 * H])
    g = jnp.tanh(gates[:, 2 * H:3 * H])
    o = jax.nn.sigmoid(gates[:, 3 * H:4 * H])
    c_new = f * c_ref[...] + i * g
    h_new = o * jnp.tanh(c_new)
    c_ref[...] = c_new
    h_ref[...] = h_new

    @pl.when(t == T - 1)
    def _():
        a = jnp.maximum(
            jnp.dot(h_new, w1_ref[...], preferred_element_type=jnp.float32)
            + b1_ref[...],
            0.0,
        )
        out_ref[...] = (
            jnp.dot(a, w2_ref[...], preferred_element_type=jnp.float32)
            + b2_ref[...]
        )


def kernel(src_seq, src_pos, emb, W_ih, W_hh, b_ih, b_hh, W1, b1, W2, b2):
    bias = (b_ih + b_hh).reshape(1, G)
    P = pl.pallas_call(
        _proj_body,
        out_shape=jax.ShapeDtypeStruct((V, G), jnp.float32),
    )(emb, W_ih.T, bias)

    flat_idx = src_seq.T.reshape(T * B).astype(jnp.int32)
    gather = pl.kernel(
        _gather_body,
        out_type=jax.ShapeDtypeStruct((T * B, G), jnp.float32),
        mesh=plsc.VectorSubcoreMesh(core_axis_name="c", subcore_axis_name="s"),
        scratch_types=[
            pltpu.VMEM((_ROWS_PER_W,), jnp.int32),
            pltpu.VMEM((_CHUNK, G), jnp.float32),
            pltpu.SemaphoreType.DMA,
        ],
    )
    X = gather(P, flat_idx).reshape(T, B, G)

    W2p = jnp.pad(W2.T, ((0, 0), (0, OUT_PAD - 2)))
    b2p = jnp.pad(b2, (0, OUT_PAD - 2)).reshape(1, OUT_PAD)

    out_p = pl.pallas_call(
        _lstm_body,
        grid=(T,),
        in_specs=[
            pl.BlockSpec((1, B, G), lambda t: (t, 0, 0)),
            pl.BlockSpec((H, G), lambda t: (0, 0)),
            pl.BlockSpec((H, F), lambda t: (0, 0)),
            pl.BlockSpec((1, F), lambda t: (0, 0)),
            pl.BlockSpec((F, OUT_PAD), lambda t: (0, 0)),
            pl.BlockSpec((1, OUT_PAD), lambda t: (0, 0)),
        ],
        out_specs=pl.BlockSpec((B, OUT_PAD), lambda t: (0, 0)),
        out_shape=jax.ShapeDtypeStruct((B, OUT_PAD), jnp.float32),
        scratch_shapes=[
            pltpu.VMEM((B, H), jnp.float32),
            pltpu.VMEM((B, H), jnp.float32),
        ],
    )(X, W_hh.T, W1.T, b1.reshape(1, F), W2p, b2p)
    return out_p[:, :2]
